# baseline (speedup 1.0000x reference)
import functools

import jax
import jax.numpy as jnp
from jax import lax
from jax.experimental import pallas as pl
from jax.experimental.pallas import tpu as pltpu
from jax.experimental.pallas import tpu_sc as plsc

B = 8192
D_IN = 2048
H = 128
H2 = 64
D_OUT = 1024
E = 8
GH = 64

BT = 512
TS = 256
NA = 2 * B
G = NA // TS + E
PPAD = G * TS
NW = 32
CH = NA // NW
SUB = 32
SUBD = 64


def _gate_kernel(x_ref, gW1_ref, gb1_ref, gW2_ref, gb2_ref,
                 xb_ref, ti_ref, tw_ref):
    xt = x_ref[:]
    gh = jnp.maximum(
        jnp.dot(xt, gW1_ref[:], preferred_element_type=jnp.float32)
        + gb1_ref[:][None, :], 0.0)
    logits = jnp.dot(gh, gW2_ref[:], preferred_element_type=jnp.float32) \
        + gb2_ref[:][None, :]
    eids = lax.broadcasted_iota(jnp.int32, (BT, E), 1)
    i1 = jnp.argmax(logits, axis=-1).astype(jnp.int32)
    v1 = jnp.max(logits, axis=-1)
    masked = jnp.where(eids == i1[:, None], -jnp.inf, logits)
    i2 = jnp.argmax(masked, axis=-1).astype(jnp.int32)
    v2 = jnp.max(masked, axis=-1)
    g1 = jax.nn.sigmoid(v1 - v2)
    xb_ref[:] = xt.astype(jnp.bfloat16)
    ti_ref[:] = jnp.concatenate([i1[:, None], i2[:, None]], axis=1)
    tw_ref[:] = jnp.concatenate([g1[:, None], (1.0 - g1)[:, None]], axis=1)


def _route_kernel(e2_ref, pos_ref, te_ref):
    ef = e2_ref[:]
    r = lax.broadcasted_iota(jnp.int32, (128, 128), 0)
    c = lax.broadcasted_iota(jnp.int32, (128, 128), 1)
    t_incl = (r <= c).astype(jnp.bfloat16)
    l_strict = (c < r).astype(jnp.float32)
    withins, sums = [], []
    for e in range(E):
        ae = (ef == e).astype(jnp.bfloat16)
        w = jnp.dot(ae, t_incl, preferred_element_type=jnp.float32)
        withins.append(w)
        sums.append(w[:, 127:128])
    s = jnp.concatenate(sums, axis=1)
    cp = jnp.dot(l_strict, s, preferred_element_type=jnp.float32)
    totals = cp[127:128, :] + s[127:128, :]
    pad = jnp.floor((totals + (TS - 1)) / TS) * TS
    posf = jnp.zeros((128, 128), dtype=jnp.float32)
    run = jnp.zeros((1, 1), dtype=jnp.float32)
    gi = lax.broadcasted_iota(jnp.int32, (1, 128), 1).astype(jnp.float32) * TS
    te_acc = jnp.zeros((1, 128), dtype=jnp.float32)
    for e in range(E):
        start = run
        run = run + pad[:, e:e + 1]
        ae = (ef == e).astype(jnp.float32)
        rank_incl = withins[e] + cp[:, e:e + 1]
        posf = posf + ae * (rank_incl - 1.0 + start)
        te_acc = te_acc + (gi >= run).astype(jnp.float32)
    pos_ref[:] = posf.astype(jnp.int32)
    te_ref[:] = jnp.minimum(te_acc, float(E - 1)).astype(jnp.int32)


def _dispatch_body(xb_hbm, pos_hbm, xs_hbm, tokv, posv, buf, sem):
    wid = lax.axis_index("s") * 2 + lax.axis_index("c")
    base = wid * CH
    iota = lax.iota(jnp.int32, 16)
    for v in range(CH // 16):
        tokv[pl.ds(v * 16, 16)] = lax.shift_right_logical(
            iota + (base + v * 16), 1)
    pltpu.sync_copy(pos_hbm.at[wid], posv)
    for sub in range(CH // SUB):
        bslot = sub % 2
        pltpu.async_copy(xb_hbm.at[tokv.at[pl.ds(sub * SUB, SUB)]],
                         buf.at[bslot], sem).wait()
        pltpu.async_copy(buf.at[bslot], xs_hbm.at[posv.at[sub]], sem).wait()


def _dispatch(xb3, pos3):
    mesh = plsc.VectorSubcoreMesh(core_axis_name="c", subcore_axis_name="s")
    return pl.kernel(
        _dispatch_body,
        jax.ShapeDtypeStruct((PPAD, 8, 128), jnp.int32),
        mesh=mesh,
        scratch_types=[
            pltpu.VMEM((CH,), jnp.int32),
            pltpu.VMEM((CH // SUB, SUB), jnp.int32),
            pltpu.VMEM((2, SUB, 8, 128), jnp.int32),
            pltpu.SemaphoreType.DMA,
        ],
    )(xb3, pos3)


def _ffn_kernel(te_ref, xs_ref, W1_ref, b1_ref, W2_ref, b2_ref,
                W3_ref, b3_ref, ys_ref):
    e = te_ref[pl.program_id(0)]
    xt = xs_ref[:]
    h1 = jnp.maximum(
        jnp.dot(xt, W1_ref[e], preferred_element_type=jnp.float32)
        + b1_ref[e][None, :], 0.0)
    h2 = jnp.maximum(
        jnp.dot(h1.astype(jnp.bfloat16), W2_ref[e],
                preferred_element_type=jnp.float32)
        + b2_ref[e][None, :], 0.0)
    y = jnp.dot(h2.astype(jnp.bfloat16), W3_ref[e],
                preferred_element_type=jnp.float32) + b3_ref[e][None, :]
    ys_ref[:] = y.astype(jnp.bfloat16)


def _ungather_body(ys_hbm, pos_hbm, ya_hbm, posv, buf, sem):
    wid = lax.axis_index("s") * 2 + lax.axis_index("c")
    base = wid * CH
    pltpu.sync_copy(pos_hbm.at[pl.ds(base, CH)], posv)
    for sub in range(CH // SUBD):
        bslot = sub % 2
        pltpu.async_copy(ys_hbm.at[posv.at[pl.ds(sub * SUBD, SUBD)]],
                         buf.at[bslot], sem).wait()
        pltpu.sync_copy(buf.at[bslot], ya_hbm.at[pl.ds(base + sub * SUBD, SUBD)])


def _ungather(ys3, pos1):
    mesh = plsc.VectorSubcoreMesh(core_axis_name="c", subcore_axis_name="s")
    return pl.kernel(
        _ungather_body,
        jax.ShapeDtypeStruct((NA, 4, 128), jnp.int32),
        mesh=mesh,
        scratch_types=[
            pltpu.VMEM((CH,), jnp.int32),
            pltpu.VMEM((2, SUBD, 4, 128), jnp.int32),
            pltpu.SemaphoreType.DMA,
        ],
    )(ys3, pos1)


def _combine_kernel(ya_ref, tw_ref, out_ref):
    ya = ya_ref[:]
    w = tw_ref[:]
    out_ref[:] = w[:, 0:1] * ya[:, :D_OUT].astype(jnp.float32) \
        + w[:, 1:2] * ya[:, D_OUT:].astype(jnp.float32)


@jax.jit
def kernel(x, gW1, gb1, gW2, gb2, W1, b1, W2, b2, W3, b3):
    full = lambda shape: pl.BlockSpec(shape, lambda i: (0,) * len(shape))
    xb, ti, tw = pl.pallas_call(
        _gate_kernel,
        grid=(B // BT,),
        in_specs=[pl.BlockSpec((BT, D_IN), lambda i: (i, 0)),
                  full((D_IN, GH)), full((GH,)), full((GH, E)), full((E,))],
        out_specs=[pl.BlockSpec((BT, D_IN), lambda i: (i, 0)),
                   pl.BlockSpec((BT, 2), lambda i: (i, 0)),
                   pl.BlockSpec((BT, 2), lambda i: (i, 0))],
        out_shape=[jax.ShapeDtypeStruct((B, D_IN), jnp.bfloat16),
                   jax.ShapeDtypeStruct((B, 2), jnp.int32),
                   jax.ShapeDtypeStruct((B, 2), jnp.float32)],
    )(x, gW1, gb1, gW2, gb2)

    e2 = ti.reshape(128, 128)
    pos2, te2 = pl.pallas_call(
        _route_kernel,
        in_specs=[pl.BlockSpec((128, 128), lambda: (0, 0))],
        out_specs=[pl.BlockSpec((128, 128), lambda: (0, 0)),
                   pl.BlockSpec((1, 128), lambda: (0, 0))],
        out_shape=[jax.ShapeDtypeStruct((128, 128), jnp.int32),
                   jax.ShapeDtypeStruct((1, 128), jnp.int32)],
    )(e2)
    te = te2[0, :G]

    xbi = lax.bitcast_convert_type(
        xb.reshape(B, D_IN // 2, 2), jnp.int32).reshape(B, 8, 128)
    xs3 = _dispatch(xbi, pos2.reshape(NW, CH // SUB, SUB))
    xsb = lax.bitcast_convert_type(
        xs3.reshape(PPAD, D_IN // 2), jnp.bfloat16).reshape(PPAD, D_IN)

    W1b = W1.astype(jnp.bfloat16)
    W2b = W2.astype(jnp.bfloat16)
    W3b = W3.astype(jnp.bfloat16)
    ys = pl.pallas_call(
        _ffn_kernel,
        grid_spec=pltpu.PrefetchScalarGridSpec(
            num_scalar_prefetch=1,
            grid=(G,),
            in_specs=[pl.BlockSpec((TS, D_IN), lambda i, te_r: (i, 0)),
                      pl.BlockSpec((E, D_IN, H), lambda i, te_r: (0, 0, 0)),
                      pl.BlockSpec((E, H), lambda i, te_r: (0, 0)),
                      pl.BlockSpec((E, H, H2), lambda i, te_r: (0, 0, 0)),
                      pl.BlockSpec((E, H2), lambda i, te_r: (0, 0)),
                      pl.BlockSpec((E, H2, D_OUT), lambda i, te_r: (0, 0, 0)),
                      pl.BlockSpec((E, D_OUT), lambda i, te_r: (0, 0))],
            out_specs=pl.BlockSpec((TS, D_OUT), lambda i, te_r: (i, 0)),
        ),
        out_shape=jax.ShapeDtypeStruct((PPAD, D_OUT), jnp.bfloat16),
    )(te, xsb, W1b, b1, W2b, b2, W3b, b3)

    ysi = lax.bitcast_convert_type(
        ys.reshape(PPAD, D_OUT // 2, 2), jnp.int32).reshape(PPAD, 4, 128)
    ya3 = _ungather(ysi, pos2.reshape(NA))
    yab = lax.bitcast_convert_type(
        ya3.reshape(NA, D_OUT // 2), jnp.bfloat16).reshape(B, 2 * D_OUT)

    out = pl.pallas_call(
        _combine_kernel,
        grid=(B // BT,),
        in_specs=[pl.BlockSpec((BT, 2 * D_OUT), lambda i: (i, 0)),
                  pl.BlockSpec((BT, 2), lambda i: (i, 0))],
        out_specs=pl.BlockSpec((BT, D_OUT), lambda i: (i, 0)),
        out_shape=jax.ShapeDtypeStruct((B, D_OUT), jnp.float32),
    )(yab, tw)
    return out


# speedup vs baseline: 25.3895x; 25.3895x over previous
import functools

import jax
import jax.numpy as jnp
from jax import lax
from jax.experimental import pallas as pl
from jax.experimental.pallas import tpu as pltpu
from jax.experimental.pallas import tpu_sc as plsc

B = 8192
D_IN = 2048
H = 128
H2 = 64
D_OUT = 1024
E = 8
GH = 64

BT = 512
TS = 256
NA = 2 * B
G = NA // TS + E
PPAD = G * TS
NW = 32
CH = NA // NW
SUB = 16
SUBD = 32


def _gate_kernel(x_ref, gW1_ref, gb1_ref, gW2_ref, gb2_ref,
                 ti_ref, tw_ref):
    xt = x_ref[:]
    gh = jnp.maximum(
        jnp.dot(xt, gW1_ref[:], preferred_element_type=jnp.float32)
        + gb1_ref[:][None, :], 0.0)
    logits = jnp.dot(gh, gW2_ref[:], preferred_element_type=jnp.float32) \
        + gb2_ref[:][None, :]
    eids = lax.broadcasted_iota(jnp.int32, (BT, E), 1)
    i1 = jnp.argmax(logits, axis=-1).astype(jnp.int32)
    v1 = jnp.max(logits, axis=-1)
    masked = jnp.where(eids == i1[:, None], -jnp.inf, logits)
    i2 = jnp.argmax(masked, axis=-1).astype(jnp.int32)
    v2 = jnp.max(masked, axis=-1)
    g1 = jax.nn.sigmoid(v1 - v2)
    ti_ref[:] = jnp.concatenate([i1[:, None], i2[:, None]], axis=1)
    tw_ref[:] = jnp.concatenate([g1[:, None], (1.0 - g1)[:, None]], axis=1)


def _route_kernel(e2_ref, pos_ref, te_ref):
    ef = e2_ref[:]
    r = lax.broadcasted_iota(jnp.int32, (128, 128), 0)
    c = lax.broadcasted_iota(jnp.int32, (128, 128), 1)
    t_incl = (r <= c).astype(jnp.bfloat16)
    l_strict = (c < r).astype(jnp.float32)
    withins, sums = [], []
    for e in range(E):
        ae = (ef == e).astype(jnp.bfloat16)
        w = jnp.dot(ae, t_incl, preferred_element_type=jnp.float32)
        withins.append(w)
        sums.append(w[:, 127:128])
    s = jnp.concatenate(sums, axis=1)
    cp = jnp.dot(l_strict, s, preferred_element_type=jnp.float32)
    totals = cp[127:128, :] + s[127:128, :]
    pad = jnp.floor((totals + (TS - 1)) / TS) * TS
    posf = jnp.zeros((128, 128), dtype=jnp.float32)
    run = jnp.zeros((1, 1), dtype=jnp.float32)
    gi = lax.broadcasted_iota(jnp.int32, (1, 128), 1).astype(jnp.float32) * TS
    te_acc = jnp.zeros((1, 128), dtype=jnp.float32)
    for e in range(E):
        start = run
        run = run + pad[:, e:e + 1]
        ae = (ef == e).astype(jnp.float32)
        rank_incl = withins[e] + cp[:, e:e + 1]
        posf = posf + ae * (rank_incl - 1.0 + start)
        te_acc = te_acc + (gi >= run).astype(jnp.float32)
    pos_ref[:] = posf.astype(jnp.int32)
    te_ref[:] = jnp.minimum(te_acc, float(E - 1)).astype(jnp.int32)


def _dispatch_body(x_hbm, tok_hbm, pos_hbm, xs_hbm, tokv, posv, buf, sem):
    wid = lax.axis_index("s") * 2 + lax.axis_index("c")
    pltpu.sync_copy(tok_hbm.at[wid], tokv)
    pltpu.sync_copy(pos_hbm.at[wid], posv)

    def step(sub, _):
        pltpu.async_copy(x_hbm.at[tokv.at[pl.ds(sub * SUB, SUB)]],
                         buf.at[0], sem).wait()
        pltpu.async_copy(buf.at[0], xs_hbm.at[posv.at[sub]], sem).wait()
        return 0

    lax.fori_loop(0, CH // SUB, step, 0)


def _dispatch(x, tok2, pos3):
    mesh = plsc.VectorSubcoreMesh(core_axis_name="c", subcore_axis_name="s")
    return pl.kernel(
        _dispatch_body,
        jax.ShapeDtypeStruct((PPAD, D_IN), jnp.float32),
        mesh=mesh,
        scratch_types=[
            pltpu.VMEM((CH,), jnp.int32),
            pltpu.VMEM((CH // SUB, SUB), jnp.int32),
            pltpu.VMEM((1, SUB, D_IN), jnp.float32),
            pltpu.SemaphoreType.DMA,
        ],
    )(x, tok2, pos3)


def _ffn_kernel(te_ref, xs_ref, W1_ref, b1_ref, W2_ref, b2_ref,
                W3_ref, b3_ref, ys_ref):
    e = te_ref[pl.program_id(0)]
    xt = xs_ref[:].astype(jnp.bfloat16)
    h1 = jnp.maximum(
        jnp.dot(xt, W1_ref[e], preferred_element_type=jnp.float32)
        + b1_ref[e][None, :], 0.0)
    h2 = jnp.maximum(
        jnp.dot(h1.astype(jnp.bfloat16), W2_ref[e],
                preferred_element_type=jnp.float32)
        + b2_ref[e][None, :], 0.0)
    y = jnp.dot(h2.astype(jnp.bfloat16), W3_ref[e],
                preferred_element_type=jnp.float32) + b3_ref[e][None, :]
    ys_ref[:] = y


def _ungather_body(ys_hbm, pos_hbm, ya_hbm, posv, buf, sem):
    wid = lax.axis_index("s") * 2 + lax.axis_index("c")
    base = wid * CH
    pltpu.sync_copy(pos_hbm.at[pl.ds(base, CH)], posv)

    def step(sub, _):
        pltpu.async_copy(ys_hbm.at[posv.at[pl.ds(sub * SUBD, SUBD)]],
                         buf.at[0], sem).wait()
        pltpu.sync_copy(buf.at[0], ya_hbm.at[pl.ds(base + sub * SUBD, SUBD)])
        return 0

    lax.fori_loop(0, CH // SUBD, step, 0)


def _ungather(ys, pos1):
    mesh = plsc.VectorSubcoreMesh(core_axis_name="c", subcore_axis_name="s")
    return pl.kernel(
        _ungather_body,
        jax.ShapeDtypeStruct((NA, D_OUT), jnp.float32),
        mesh=mesh,
        scratch_types=[
            pltpu.VMEM((CH,), jnp.int32),
            pltpu.VMEM((1, SUBD, D_OUT), jnp.float32),
            pltpu.SemaphoreType.DMA,
        ],
    )(ys, pos1)


def _combine_kernel(y0_ref, y1_ref, tw_ref, out_ref):
    w = tw_ref[:]
    out_ref[:] = w[:, 0:1] * y0_ref[:] + w[:, 1:2] * y1_ref[:]


@jax.jit
def kernel(x, gW1, gb1, gW2, gb2, W1, b1, W2, b2, W3, b3):
    full = lambda shape: pl.BlockSpec(shape, lambda i: (0,) * len(shape))
    ti, tw = pl.pallas_call(
        _gate_kernel,
        grid=(B // BT,),
        in_specs=[pl.BlockSpec((BT, D_IN), lambda i: (i, 0)),
                  full((D_IN, GH)), full((GH,)), full((GH, E)), full((E,))],
        out_specs=[pl.BlockSpec((BT, 2), lambda i: (i, 0)),
                   pl.BlockSpec((BT, 2), lambda i: (i, 0))],
        out_shape=[jax.ShapeDtypeStruct((B, 2), jnp.int32),
                   jax.ShapeDtypeStruct((B, 2), jnp.float32)],
    )(x, gW1, gb1, gW2, gb2)

    e2 = jnp.concatenate([ti[:, 0], ti[:, 1]]).reshape(128, 128)
    pos2, te2 = pl.pallas_call(
        _route_kernel,
        in_specs=[pl.BlockSpec((128, 128), lambda: (0, 0))],
        out_specs=[pl.BlockSpec((128, 128), lambda: (0, 0)),
                   pl.BlockSpec((1, 128), lambda: (0, 0))],
        out_shape=[jax.ShapeDtypeStruct((128, 128), jnp.int32),
                   jax.ShapeDtypeStruct((1, 128), jnp.int32)],
    )(e2)
    te = te2[0, :G]

    tok2 = (jnp.arange(NA, dtype=jnp.int32) % B).reshape(NW, CH)
    xs = _dispatch(x, tok2, pos2.reshape(NW, CH // SUB, SUB))

    W1b = W1.astype(jnp.bfloat16)
    W2b = W2.astype(jnp.bfloat16)
    W3b = W3.astype(jnp.bfloat16)
    ys = pl.pallas_call(
        _ffn_kernel,
        grid_spec=pltpu.PrefetchScalarGridSpec(
            num_scalar_prefetch=1,
            grid=(G,),
            in_specs=[pl.BlockSpec((TS, D_IN), lambda i, te_r: (i, 0)),
                      pl.BlockSpec((E, D_IN, H), lambda i, te_r: (0, 0, 0)),
                      pl.BlockSpec((E, H), lambda i, te_r: (0, 0)),
                      pl.BlockSpec((E, H, H2), lambda i, te_r: (0, 0, 0)),
                      pl.BlockSpec((E, H2), lambda i, te_r: (0, 0)),
                      pl.BlockSpec((E, H2, D_OUT), lambda i, te_r: (0, 0, 0)),
                      pl.BlockSpec((E, D_OUT), lambda i, te_r: (0, 0))],
            out_specs=pl.BlockSpec((TS, D_OUT), lambda i, te_r: (i, 0)),
        ),
        out_shape=jax.ShapeDtypeStruct((PPAD, D_OUT), jnp.float32),
    )(te, xs, W1b, b1, W2b, b2, W3b, b3)

    ya = _ungather(ys, pos2.reshape(NA))

    nb = B // BT
    out = pl.pallas_call(
        _combine_kernel,
        grid=(nb,),
        in_specs=[pl.BlockSpec((BT, D_OUT), lambda i: (i, 0)),
                  pl.BlockSpec((BT, D_OUT), lambda i: (i + nb, 0)),
                  pl.BlockSpec((BT, 2), lambda i: (i, 0))],
        out_specs=pl.BlockSpec((BT, D_OUT), lambda i: (i, 0)),
        out_shape=jax.ShapeDtypeStruct((B, D_OUT), jnp.float32),
    )(ya, ya, tw)
    return out


# speedup vs baseline: 27.7325x; 1.0923x over previous
import functools

import jax
import jax.numpy as jnp
from jax import lax
from jax.experimental import pallas as pl
from jax.experimental.pallas import tpu as pltpu
from jax.experimental.pallas import tpu_sc as plsc

B = 8192
D_IN = 2048
H = 128
H2 = 64
D_OUT = 1024
E = 8
GH = 64

BT = 512
TS = 256
NA = 2 * B
G = NA // TS + E
PPAD = G * TS
NW = 32
CH = NA // NW
SUB = 16
SUBD = 32


def _gate_kernel(x_ref, gW1_ref, gb1_ref, gW2_ref, gb2_ref,
                 ti_ref, tw_ref):
    xt = x_ref[:]
    gh = jnp.maximum(
        jnp.dot(xt, gW1_ref[:], preferred_element_type=jnp.float32)
        + gb1_ref[:][None, :], 0.0)
    logits = jnp.dot(gh, gW2_ref[:], preferred_element_type=jnp.float32) \
        + gb2_ref[:][None, :]
    eids = lax.broadcasted_iota(jnp.int32, (BT, E), 1)
    i1 = jnp.argmax(logits, axis=-1).astype(jnp.int32)
    v1 = jnp.max(logits, axis=-1)
    masked = jnp.where(eids == i1[:, None], -jnp.inf, logits)
    i2 = jnp.argmax(masked, axis=-1).astype(jnp.int32)
    v2 = jnp.max(masked, axis=-1)
    g1 = jax.nn.sigmoid(v1 - v2)
    ti_ref[:] = jnp.concatenate([i1[:, None], i2[:, None]], axis=1)
    tw_ref[:] = jnp.concatenate([g1[:, None], (1.0 - g1)[:, None]], axis=1)


def _route_kernel(e2_ref, pos_ref, te_ref):
    ef = e2_ref[:]
    r = lax.broadcasted_iota(jnp.int32, (128, 128), 0)
    c = lax.broadcasted_iota(jnp.int32, (128, 128), 1)
    t_incl = (r <= c).astype(jnp.bfloat16)
    l_strict = (c < r).astype(jnp.float32)
    withins, sums = [], []
    for e in range(E):
        ae = (ef == e).astype(jnp.bfloat16)
        w = jnp.dot(ae, t_incl, preferred_element_type=jnp.float32)
        withins.append(w)
        sums.append(w[:, 127:128])
    s = jnp.concatenate(sums, axis=1)
    cp = jnp.dot(l_strict, s, preferred_element_type=jnp.float32)
    totals = cp[127:128, :] + s[127:128, :]
    pad = jnp.floor((totals + (TS - 1)) / TS) * TS
    posf = jnp.zeros((128, 128), dtype=jnp.float32)
    run = jnp.zeros((1, 1), dtype=jnp.float32)
    gi = lax.broadcasted_iota(jnp.int32, (1, 128), 1).astype(jnp.float32) * TS
    te_acc = jnp.zeros((1, 128), dtype=jnp.float32)
    for e in range(E):
        start = run
        run = run + pad[:, e:e + 1]
        ae = (ef == e).astype(jnp.float32)
        rank_incl = withins[e] + cp[:, e:e + 1]
        posf = posf + ae * (rank_incl - 1.0 + start)
        te_acc = te_acc + (gi >= run).astype(jnp.float32)
    pos_ref[:] = posf.astype(jnp.int32)
    te_ref[:] = jnp.minimum(te_acc, float(E - 1)).astype(jnp.int32)


def _dispatch_body(x_hbm, tok_hbm, pos_hbm, xs_hbm, tokv, posv, buf,
                   semg, semsc):
    wid = lax.axis_index("s") * 2 + lax.axis_index("c")
    pltpu.sync_copy(tok_hbm.at[wid], tokv)
    pltpu.sync_copy(pos_hbm.at[wid], posv)
    nstep = CH // SUB

    def start_g(s, slot):
        pltpu.async_copy(x_hbm.at[tokv.at[pl.ds(s * SUB, SUB)]],
                         buf.at[slot], semg)

    def wait_g():
        pltpu.make_async_copy(x_hbm.at[tokv.at[pl.ds(0, SUB)]],
                              buf.at[0], semg).wait()

    def start_s(s, slot):
        pltpu.async_copy(buf.at[slot], xs_hbm.at[posv.at[s]], semsc)

    def wait_s():
        pltpu.make_async_copy(buf.at[0], xs_hbm.at[posv.at[0]], semsc).wait()

    start_g(0, 0)

    def body(t, _):
        for u in range(2):
            s = 2 * t + u

            @pl.when(s >= 1)
            def _():
                wait_s()

            @pl.when(s + 1 < nstep)
            def _():
                start_g(s + 1, 1 - u)

            wait_g()
            start_s(s, u)
        return 0

    lax.fori_loop(0, nstep // 2, body, 0)
    wait_s()


def _dispatch(x, tok2, pos3):
    mesh = plsc.VectorSubcoreMesh(core_axis_name="c", subcore_axis_name="s")
    return pl.kernel(
        _dispatch_body,
        jax.ShapeDtypeStruct((PPAD, D_IN), jnp.float32),
        mesh=mesh,
        scratch_types=[
            pltpu.VMEM((CH,), jnp.int32),
            pltpu.VMEM((CH // SUB, SUB), jnp.int32),
            pltpu.VMEM((2, SUB, D_IN), jnp.float32),
            pltpu.SemaphoreType.DMA,
            pltpu.SemaphoreType.DMA,
        ],
    )(x, tok2, pos3)


def _ffn_kernel(te_ref, xs_ref, W1_ref, b1_ref, W2_ref, b2_ref,
                W3_ref, b3_ref, ys_ref):
    e = te_ref[pl.program_id(0)]
    xt = xs_ref[:].astype(jnp.bfloat16)
    h1 = jnp.maximum(
        jnp.dot(xt, W1_ref[e], preferred_element_type=jnp.float32)
        + b1_ref[e][None, :], 0.0)
    h2 = jnp.maximum(
        jnp.dot(h1.astype(jnp.bfloat16), W2_ref[e],
                preferred_element_type=jnp.float32)
        + b2_ref[e][None, :], 0.0)
    y = jnp.dot(h2.astype(jnp.bfloat16), W3_ref[e],
                preferred_element_type=jnp.float32) + b3_ref[e][None, :]
    ys_ref[:] = y


def _ungather_body(ys_hbm, pos_hbm, ya_hbm, posv, buf, semg, semw):
    wid = lax.axis_index("s") * 2 + lax.axis_index("c")
    base = wid * CH
    pltpu.sync_copy(pos_hbm.at[pl.ds(base, CH)], posv)
    nstep = CH // SUBD

    def start_g(s, slot):
        pltpu.async_copy(ys_hbm.at[posv.at[pl.ds(s * SUBD, SUBD)]],
                         buf.at[slot], semg)

    def wait_g():
        pltpu.make_async_copy(ys_hbm.at[posv.at[pl.ds(0, SUBD)]],
                              buf.at[0], semg).wait()

    def start_w(s, slot):
        pltpu.async_copy(buf.at[slot],
                         ya_hbm.at[pl.ds(base + s * SUBD, SUBD)], semw)

    def wait_w():
        pltpu.make_async_copy(buf.at[0], ya_hbm.at[pl.ds(base, SUBD)],
                              semw).wait()

    start_g(0, 0)

    def body(t, _):
        for u in range(2):
            s = 2 * t + u

            @pl.when(s >= 1)
            def _():
                wait_w()

            @pl.when(s + 1 < nstep)
            def _():
                start_g(s + 1, 1 - u)

            wait_g()
            start_w(s, u)
        return 0

    lax.fori_loop(0, nstep // 2, body, 0)
    wait_w()


def _ungather(ys, pos1):
    mesh = plsc.VectorSubcoreMesh(core_axis_name="c", subcore_axis_name="s")
    return pl.kernel(
        _ungather_body,
        jax.ShapeDtypeStruct((NA, D_OUT), jnp.float32),
        mesh=mesh,
        scratch_types=[
            pltpu.VMEM((CH,), jnp.int32),
            pltpu.VMEM((2, SUBD, D_OUT), jnp.float32),
            pltpu.SemaphoreType.DMA,
            pltpu.SemaphoreType.DMA,
        ],
    )(ys, pos1)


def _combine_kernel(y0_ref, y1_ref, tw_ref, out_ref):
    w = tw_ref[:]
    out_ref[:] = w[:, 0:1] * y0_ref[:] + w[:, 1:2] * y1_ref[:]


@jax.jit
def kernel(x, gW1, gb1, gW2, gb2, W1, b1, W2, b2, W3, b3):
    full = lambda shape: pl.BlockSpec(shape, lambda i: (0,) * len(shape))
    ti, tw = pl.pallas_call(
        _gate_kernel,
        grid=(B // BT,),
        in_specs=[pl.BlockSpec((BT, D_IN), lambda i: (i, 0)),
                  full((D_IN, GH)), full((GH,)), full((GH, E)), full((E,))],
        out_specs=[pl.BlockSpec((BT, 2), lambda i: (i, 0)),
                   pl.BlockSpec((BT, 2), lambda i: (i, 0))],
        out_shape=[jax.ShapeDtypeStruct((B, 2), jnp.int32),
                   jax.ShapeDtypeStruct((B, 2), jnp.float32)],
    )(x, gW1, gb1, gW2, gb2)

    e2 = jnp.concatenate([ti[:, 0], ti[:, 1]]).reshape(128, 128)
    pos2, te2 = pl.pallas_call(
        _route_kernel,
        in_specs=[pl.BlockSpec((128, 128), lambda: (0, 0))],
        out_specs=[pl.BlockSpec((128, 128), lambda: (0, 0)),
                   pl.BlockSpec((1, 128), lambda: (0, 0))],
        out_shape=[jax.ShapeDtypeStruct((128, 128), jnp.int32),
                   jax.ShapeDtypeStruct((1, 128), jnp.int32)],
    )(e2)
    te = te2[0, :G]

    tok2 = (jnp.arange(NA, dtype=jnp.int32) % B).reshape(NW, CH)
    xs = _dispatch(x, tok2, pos2.reshape(NW, CH // SUB, SUB))

    W1b = W1.astype(jnp.bfloat16)
    W2b = W2.astype(jnp.bfloat16)
    W3b = W3.astype(jnp.bfloat16)
    ys = pl.pallas_call(
        _ffn_kernel,
        grid_spec=pltpu.PrefetchScalarGridSpec(
            num_scalar_prefetch=1,
            grid=(G,),
            in_specs=[pl.BlockSpec((TS, D_IN), lambda i, te_r: (i, 0)),
                      pl.BlockSpec((E, D_IN, H), lambda i, te_r: (0, 0, 0)),
                      pl.BlockSpec((E, H), lambda i, te_r: (0, 0)),
                      pl.BlockSpec((E, H, H2), lambda i, te_r: (0, 0, 0)),
                      pl.BlockSpec((E, H2), lambda i, te_r: (0, 0)),
                      pl.BlockSpec((E, H2, D_OUT), lambda i, te_r: (0, 0, 0)),
                      pl.BlockSpec((E, D_OUT), lambda i, te_r: (0, 0))],
            out_specs=pl.BlockSpec((TS, D_OUT), lambda i, te_r: (i, 0)),
        ),
        out_shape=jax.ShapeDtypeStruct((PPAD, D_OUT), jnp.float32),
    )(te, xs, W1b, b1, W2b, b2, W3b, b3)

    ya = _ungather(ys, pos2.reshape(NA))

    nb = B // BT
    out = pl.pallas_call(
        _combine_kernel,
        grid=(nb,),
        in_specs=[pl.BlockSpec((BT, D_OUT), lambda i: (i, 0)),
                  pl.BlockSpec((BT, D_OUT), lambda i: (i + nb, 0)),
                  pl.BlockSpec((BT, 2), lambda i: (i, 0))],
        out_specs=pl.BlockSpec((BT, D_OUT), lambda i: (i, 0)),
        out_shape=jax.ShapeDtypeStruct((B, D_OUT), jnp.float32),
    )(ya, ya, tw)
    return out


# speedup vs baseline: 93.0561x; 3.3555x over previous
import functools

import jax
import jax.numpy as jnp
from jax.experimental import pallas as pl

B = 8192
D_IN = 2048
H = 128
H2 = 64
D_OUT = 1024
E = 8
GH = 64
TOP_K = 2

BT = 512


def _moe_kernel(x_ref, gW1_ref, gb1_ref, gW2_ref, gb2_ref,
                W1_ref, b1_ref, W2_ref, b2_ref, W3_ref, b3_ref, out_ref):
    xt = x_ref[:]

    gh = jnp.maximum(
        jnp.dot(xt, gW1_ref[:], preferred_element_type=jnp.float32)
        + gb1_ref[:][None, :], 0.0)
    logits = jnp.dot(gh, gW2_ref[:], preferred_element_type=jnp.float32) \
        + gb2_ref[:][None, :]

    eids = jax.lax.broadcasted_iota(jnp.int32, (BT, E), 1)
    i1 = jnp.argmax(logits, axis=-1).astype(jnp.int32)
    v1 = jnp.max(logits, axis=-1)
    masked = jnp.where(eids == i1[:, None], -jnp.inf, logits)
    i2 = jnp.argmax(masked, axis=-1).astype(jnp.int32)
    v2 = jnp.max(masked, axis=-1)
    g1 = jax.nn.sigmoid(v1 - v2)
    g2 = 1.0 - g1
    gates = jnp.where(eids == i1[:, None], g1[:, None], 0.0) \
        + jnp.where(eids == i2[:, None], g2[:, None], 0.0)

    xb = xt.astype(jnp.bfloat16)
    h1 = jnp.dot(xb, W1_ref[:], preferred_element_type=jnp.float32)
    h1 = jnp.maximum(h1 + b1_ref[:][None, :], 0.0)
    h2s = []
    for e in range(E):
        h2 = jnp.dot(h1[:, e * H:(e + 1) * H].astype(jnp.bfloat16),
                     W2_ref[e], preferred_element_type=jnp.float32)
        h2 = jnp.maximum(h2 + b2_ref[e][None, :], 0.0)
        h2s.append(gates[:, e][:, None] * h2)
    h2cat = jnp.concatenate(h2s, axis=1)
    y = jnp.dot(h2cat.astype(jnp.bfloat16), W3_ref[:],
                preferred_element_type=jnp.float32)
    y = y + jnp.dot(gates, b3_ref[:], preferred_element_type=jnp.float32)
    out_ref[:] = y


@jax.jit
def kernel(x, gW1, gb1, gW2, gb2, W1, b1, W2, b2, W3, b3):
    W1c = jnp.transpose(W1, (1, 0, 2)).reshape(D_IN, E * H).astype(jnp.bfloat16)
    b1c = b1.reshape(E * H)
    W2b = W2.astype(jnp.bfloat16)
    W3c = W3.reshape(E * H2, D_OUT).astype(jnp.bfloat16)
    grid = (B // BT,)
    full = lambda shape: pl.BlockSpec(shape, lambda i: (0,) * len(shape))
    return pl.pallas_call(
        _moe_kernel,
        grid=grid,
        in_specs=[
            pl.BlockSpec((BT, D_IN), lambda i: (i, 0)),
            full((D_IN, GH)), full((GH,)), full((GH, E)), full((E,)),
            full((D_IN, E * H)), full((E * H,)),
            full((E, H, H2)), full((E, H2)),
            full((E * H2, D_OUT)), full((E, D_OUT)),
        ],
        out_specs=pl.BlockSpec((BT, D_OUT), lambda i: (i, 0)),
        out_shape=jax.ShapeDtypeStruct((B, D_OUT), jnp.float32),
    )(x, gW1, gb1, gW2, gb2, W1c, b1c, W2b, b2, W3c, b3)


# speedup vs baseline: 94.9413x; 1.0203x over previous
import functools

import jax
import jax.numpy as jnp
from jax.experimental import pallas as pl

B = 8192
D_IN = 2048
H = 128
H2 = 64
D_OUT = 1024
E = 8
GH = 64
TOP_K = 2

BT = 1024


def _moe_kernel(x_ref, gW1_ref, gb1_ref, gW2_ref, gb2_ref,
                W1_ref, b1_ref, W2_ref, b2_ref, W3_ref, b3_ref, out_ref):
    xt = x_ref[:]

    gh = jnp.maximum(
        jnp.dot(xt, gW1_ref[:], preferred_element_type=jnp.float32)
        + gb1_ref[:][None, :], 0.0)
    logits = jnp.dot(gh, gW2_ref[:], preferred_element_type=jnp.float32) \
        + gb2_ref[:][None, :]

    eids = jax.lax.broadcasted_iota(jnp.int32, (BT, E), 1)
    i1 = jnp.argmax(logits, axis=-1).astype(jnp.int32)
    v1 = jnp.max(logits, axis=-1)
    masked = jnp.where(eids == i1[:, None], -jnp.inf, logits)
    i2 = jnp.argmax(masked, axis=-1).astype(jnp.int32)
    v2 = jnp.max(masked, axis=-1)
    g1 = jax.nn.sigmoid(v1 - v2)
    g2 = 1.0 - g1
    gates = jnp.where(eids == i1[:, None], g1[:, None], 0.0) \
        + jnp.where(eids == i2[:, None], g2[:, None], 0.0)

    xb = xt.astype(jnp.bfloat16)
    h1 = jnp.dot(xb, W1_ref[:], preferred_element_type=jnp.float32)
    h1 = jnp.maximum(h1 + b1_ref[:][None, :], 0.0)
    h2s = []
    for e in range(E):
        h2 = jnp.dot(h1[:, e * H:(e + 1) * H].astype(jnp.bfloat16),
                     W2_ref[e], preferred_element_type=jnp.float32)
        h2 = jnp.maximum(h2 + b2_ref[e][None, :], 0.0)
        h2s.append(gates[:, e][:, None] * h2)
    h2cat = jnp.concatenate(h2s, axis=1)
    y = jnp.dot(h2cat.astype(jnp.bfloat16), W3_ref[:],
                preferred_element_type=jnp.float32)
    y = y + jnp.dot(gates, b3_ref[:], preferred_element_type=jnp.float32)
    out_ref[:] = y


@jax.jit
def kernel(x, gW1, gb1, gW2, gb2, W1, b1, W2, b2, W3, b3):
    W1c = jnp.transpose(W1, (1, 0, 2)).reshape(D_IN, E * H).astype(jnp.bfloat16)
    b1c = b1.reshape(E * H)
    W2b = W2.astype(jnp.bfloat16)
    W3c = W3.reshape(E * H2, D_OUT).astype(jnp.bfloat16)
    grid = (B // BT,)
    full = lambda shape: pl.BlockSpec(shape, lambda i: (0,) * len(shape))
    return pl.pallas_call(
        _moe_kernel,
        grid=grid,
        in_specs=[
            pl.BlockSpec((BT, D_IN), lambda i: (i, 0)),
            full((D_IN, GH)), full((GH,)), full((GH, E)), full((E,)),
            full((D_IN, E * H)), full((E * H,)),
            full((E, H, H2)), full((E, H2)),
            full((E * H2, D_OUT)), full((E, D_OUT)),
        ],
        out_specs=pl.BlockSpec((BT, D_OUT), lambda i: (i, 0)),
        out_shape=jax.ShapeDtypeStruct((B, D_OUT), jnp.float32),
    )(x, gW1, gb1, gW2, gb2, W1c, b1c, W2b, b2, W3c, b3)
